# baseline (speedup 1.0000x reference)
import functools

import jax
import jax.numpy as jnp
import numpy as np
from jax import lax
from jax.experimental import pallas as pl
from jax.experimental.pallas import tpu as pltpu
from jax.experimental.pallas import tpu_sc as plsc

NUM_BUCKETS = 32
MAX_DISTANCE = 128
NUM_HEADS = 16
S = 2048
NSHIFT = 16
LINE_LEN = 4096
LINE_PAD = LINE_LEN + NSHIFT

NUM_WORKERS = 32
ROWS_PER_WORKER = S // NUM_WORKERS


def _line_kernel(table_ref, lines_ref):
    k = lax.broadcasted_iota(jnp.int32, (NUM_HEADS, LINE_PAD), 1)
    d = k - (S - 1)
    nb = NUM_BUCKETS // 2
    rel_buckets = (d > 0).astype(jnp.int32) * nb
    ad = jnp.abs(d)
    max_exact = nb // 2
    is_small = ad < max_exact
    rp_f = jnp.maximum(ad, 1).astype(jnp.float32)
    large = max_exact + (
        jnp.log(rp_f / max_exact) / np.log(MAX_DISTANCE / max_exact) * (nb - max_exact)
    ).astype(jnp.int32)
    large = jnp.minimum(large, jnp.full_like(large, nb - 1))
    bucket = rel_buckets + jnp.where(is_small, ad, large)

    line = jnp.zeros((NUM_HEADS, LINE_PAD), jnp.float32)
    for b in range(NUM_BUCKETS):
        val = table_ref[b, :][:, None]
        line = jnp.where(bucket == b, val, line)
    for c in range(NSHIFT):
        lines_ref[c] = line[:, c : c + LINE_LEN]


def _make_sc_expand():
    mesh = plsc.VectorSubcoreMesh(core_axis_name="c", subcore_axis_name="s")

    @functools.partial(
        pl.kernel,
        mesh=mesh,
        out_type=jax.ShapeDtypeStruct((NUM_HEADS, S, S), jnp.float32),
        scratch_types=[
            pltpu.VMEM_SHARED((NSHIFT, NUM_HEADS, LINE_LEN), jnp.float32),
            pltpu.SemaphoreType.DMA,
        ],
        compiler_params=pltpu.CompilerParams(use_tc_tiling_on_sc=False),
    )
    def sc_expand(lines_hbm, out_hbm, spmem, sem):
        cid = lax.axis_index("c")
        sid = lax.axis_index("s")
        pltpu.sync_copy(lines_hbm.at[sid], spmem.at[sid])
        plsc.subcore_barrier()

        w = sid * 2 + cid

        def row_body(r, _):
            i = w * ROWS_PER_WORKER + r
            sdist = (S - 1) - i
            c16 = lax.rem(sdist, NSHIFT)
            base = pl.multiple_of(sdist - c16, NSHIFT)
            pltpu.sync_copy(
                spmem.at[c16, :, pl.ds(base, S)], out_hbm.at[:, i, :]
            )
            return 0

        lax.fori_loop(0, ROWS_PER_WORKER, row_body, 0)

    return sc_expand


def kernel(input_ids, attention_mask, bias_table):
    del input_ids, attention_mask
    lines = pl.pallas_call(
        _line_kernel,
        out_shape=jax.ShapeDtypeStruct((NSHIFT, NUM_HEADS, LINE_LEN), jnp.float32),
    )(bias_table)
    out = _make_sc_expand()(lines)
    return out[None]


# speedup vs baseline: 1.0011x; 1.0011x over previous
import functools

import jax
import jax.numpy as jnp
import numpy as np
from jax import lax
from jax.experimental import pallas as pl
from jax.experimental.pallas import tpu as pltpu
from jax.experimental.pallas import tpu_sc as plsc

NUM_BUCKETS = 32
MAX_DISTANCE = 128
NUM_HEADS = 16
S = 2048
NSHIFT = 16
LINE_LEN = 4096
LINE_PAD = LINE_LEN + NSHIFT

NUM_WORKERS = 32
ROWS_PER_WORKER = S // NUM_WORKERS


def _line_kernel(table_ref, lines_ref):
    k = lax.broadcasted_iota(jnp.int32, (NUM_HEADS, LINE_PAD), 1)
    d = k - (S - 1)
    nb = NUM_BUCKETS // 2
    rel_buckets = (d > 0).astype(jnp.int32) * nb
    ad = jnp.abs(d)
    max_exact = nb // 2
    is_small = ad < max_exact
    rp_f = jnp.maximum(ad, 1).astype(jnp.float32)
    large = max_exact + (
        jnp.log(rp_f / max_exact) / np.log(MAX_DISTANCE / max_exact) * (nb - max_exact)
    ).astype(jnp.int32)
    large = jnp.minimum(large, jnp.full_like(large, nb - 1))
    bucket = rel_buckets + jnp.where(is_small, ad, large)

    line = jnp.zeros((NUM_HEADS, LINE_PAD), jnp.float32)
    for b in range(NUM_BUCKETS):
        val = table_ref[b, :][:, None]
        line = jnp.where(bucket == b, val, line)
    for c in range(NSHIFT):
        lines_ref[c] = line[:, c : c + LINE_LEN]


def _make_sc_expand():
    mesh = plsc.VectorSubcoreMesh(core_axis_name="c", subcore_axis_name="s")

    @functools.partial(
        pl.kernel,
        mesh=mesh,
        out_type=jax.ShapeDtypeStruct((NUM_HEADS, S, S), jnp.float32),
        scratch_types=[
            pltpu.VMEM_SHARED((NSHIFT, NUM_HEADS, LINE_LEN), jnp.float32),
            pltpu.SemaphoreType.DMA,
        ],
        compiler_params=pltpu.CompilerParams(use_tc_tiling_on_sc=False),
    )
    def sc_expand(lines_hbm, out_hbm, spmem, sem):
        cid = lax.axis_index("c")
        sid = lax.axis_index("s")
        pltpu.sync_copy(lines_hbm.at[sid], spmem.at[sid])
        plsc.subcore_barrier()

        w = sid * 2 + cid
        chunk = 16

        def chunk_body(k, _):
            copies = []
            for r in range(chunk):
                i = w * ROWS_PER_WORKER + k * chunk + r
                sdist = (S - 1) - i
                c16 = lax.rem(sdist, NSHIFT)
                base = pl.multiple_of(sdist - c16, NSHIFT)
                cp = pltpu.make_async_copy(
                    spmem.at[c16, :, pl.ds(base, S)], out_hbm.at[:, i, :], sem
                )
                cp.start()
                copies.append(cp)
            for cp in copies:
                cp.wait()
            return 0

        lax.fori_loop(0, ROWS_PER_WORKER // chunk, chunk_body, 0)

    return sc_expand


def kernel(input_ids, attention_mask, bias_table):
    del input_ids, attention_mask
    lines = pl.pallas_call(
        _line_kernel,
        out_shape=jax.ShapeDtypeStruct((NSHIFT, NUM_HEADS, LINE_LEN), jnp.float32),
    )(bias_table)
    out = _make_sc_expand()(lines)
    return out[None]


# speedup vs baseline: 4.6166x; 4.6115x over previous
import functools

import jax
import jax.numpy as jnp
import numpy as np
from jax import lax
from jax.experimental import pallas as pl
from jax.experimental.pallas import tpu as pltpu

NUM_BUCKETS = 32
MAX_DISTANCE = 128
NUM_HEADS = 16
S = 2048
ROW_BLOCK = 128
J_BLOCK = 1024
LINE_LEN = 3968
LINE_PAD = LINE_LEN + ROW_BLOCK


def _compute_line():
    k = lax.broadcasted_iota(jnp.int32, (NUM_HEADS, LINE_PAD), 1)
    d = k - (S - 1)
    nb = NUM_BUCKETS // 2
    rel_buckets = (d > 0).astype(jnp.int32) * nb
    ad = jnp.abs(d)
    max_exact = nb // 2
    is_small = ad < max_exact
    rp_f = jnp.maximum(ad, 1).astype(jnp.float32)
    large = max_exact + (
        jnp.log(rp_f / max_exact) / np.log(MAX_DISTANCE / max_exact) * (nb - max_exact)
    ).astype(jnp.int32)
    large = jnp.minimum(large, jnp.full_like(large, nb - 1))
    return rel_buckets + jnp.where(is_small, ad, large)


def _fused_kernel(table_ref, out_ref, skew_ref):
    @pl.when((pl.program_id(0) == 0) & (pl.program_id(1) == 0))
    def _build():
        bucket = _compute_line()
        line = jnp.zeros((NUM_HEADS, LINE_PAD), jnp.float32)
        for b in range(NUM_BUCKETS):
            val = table_ref[b, :][:, None]
            line = jnp.where(bucket == b, val, line)
        for r in range(ROW_BLOCK):
            sh = (ROW_BLOCK - 1) - r
            skew_ref[:, r, :] = line[:, sh : sh + LINE_LEN]

    g = pl.program_id(0)
    jb = pl.program_id(1)
    off = pl.multiple_of(
        (S - ROW_BLOCK) - ROW_BLOCK * g + J_BLOCK * jb, ROW_BLOCK
    )
    out_ref[...] = skew_ref[:, :, pl.ds(off, J_BLOCK)]


def kernel(input_ids, attention_mask, bias_table):
    del input_ids, attention_mask
    out = pl.pallas_call(
        _fused_kernel,
        grid=(S // ROW_BLOCK, S // J_BLOCK),
        in_specs=[pl.BlockSpec((NUM_BUCKETS, NUM_HEADS), lambda g, jb: (0, 0))],
        out_specs=pl.BlockSpec(
            (NUM_HEADS, ROW_BLOCK, J_BLOCK), lambda g, jb: (0, g, jb)
        ),
        out_shape=jax.ShapeDtypeStruct((NUM_HEADS, S, S), jnp.float32),
        scratch_shapes=[pltpu.VMEM((NUM_HEADS, ROW_BLOCK, LINE_LEN), jnp.float32)],
    )(bias_table)
    return out[None]


# speedup vs baseline: 4.6707x; 1.0117x over previous
import functools

import jax
import jax.numpy as jnp
import numpy as np
from jax import lax
from jax.experimental import pallas as pl
from jax.experimental.pallas import tpu as pltpu

NUM_BUCKETS = 32
MAX_DISTANCE = 128
NUM_HEADS = 16
S = 2048
SKEW = 128
ROW_BLOCK = 64
LINE_LEN = 3968
LINE_PAD = LINE_LEN + SKEW


def _compute_line():
    k = lax.broadcasted_iota(jnp.int32, (NUM_HEADS, LINE_PAD), 1)
    d = k - (S - 1)
    nb = NUM_BUCKETS // 2
    rel_buckets = (d > 0).astype(jnp.int32) * nb
    ad = jnp.abs(d)
    max_exact = nb // 2
    is_small = ad < max_exact
    rp_f = jnp.maximum(ad, 1).astype(jnp.float32)
    large = max_exact + (
        jnp.log(rp_f / max_exact) / np.log(MAX_DISTANCE / max_exact) * (nb - max_exact)
    ).astype(jnp.int32)
    large = jnp.minimum(large, jnp.full_like(large, nb - 1))
    return rel_buckets + jnp.where(is_small, ad, large)


def _fused_kernel(table_ref, out_ref, skew_ref):
    g = pl.program_id(0)
    for half in range(2):

        @pl.when(g == half)
        def _build():
            bucket = _compute_line()
            line = jnp.zeros((NUM_HEADS, LINE_PAD), jnp.float32)
            for b in range(NUM_BUCKETS):
                val = table_ref[b, :][:, None]
                line = jnp.where(bucket == b, val, line)
            for r in range(half * ROW_BLOCK, (half + 1) * ROW_BLOCK):
                sh = (SKEW - 1) - r
                skew_ref[:, r, :] = line[:, sh : sh + LINE_LEN]

    off = pl.multiple_of((S - SKEW) - SKEW * (g // 2), SKEW)
    rstart = pl.multiple_of(ROW_BLOCK * lax.rem(g, 2), ROW_BLOCK)
    out_ref[...] = skew_ref[:, pl.ds(rstart, ROW_BLOCK), pl.ds(off, S)]


def kernel(input_ids, attention_mask, bias_table):
    del input_ids, attention_mask
    out = pl.pallas_call(
        _fused_kernel,
        grid=(S // ROW_BLOCK,),
        in_specs=[pl.BlockSpec((NUM_BUCKETS, NUM_HEADS), lambda g: (0, 0))],
        out_specs=pl.BlockSpec((NUM_HEADS, ROW_BLOCK, S), lambda g: (0, g, 0)),
        out_shape=jax.ShapeDtypeStruct((NUM_HEADS, S, S), jnp.float32),
        scratch_shapes=[pltpu.VMEM((NUM_HEADS, SKEW, LINE_LEN), jnp.float32)],
    )(bias_table)
    return out[None]


# speedup vs baseline: 4.9134x; 1.0520x over previous
import functools

import jax
import jax.numpy as jnp
import numpy as np
from jax import lax
from jax.experimental import pallas as pl
from jax.experimental.pallas import tpu as pltpu

NUM_BUCKETS = 32
MAX_DISTANCE = 128
NUM_HEADS = 16
S = 2048
SKEW = 128
ROW_BLOCK = 64
LINE_LEN = 3968
LINE_PAD = LINE_LEN + SKEW


def _compute_line():
    k = lax.broadcasted_iota(jnp.int32, (NUM_HEADS, LINE_PAD), 1)
    d = k - (S - 1)
    nb = NUM_BUCKETS // 2
    rel_buckets = (d > 0).astype(jnp.int32) * nb
    ad = jnp.abs(d)
    max_exact = nb // 2
    is_small = ad < max_exact
    rp_f = jnp.maximum(ad, 1).astype(jnp.float32)
    large = max_exact + (
        jnp.log(rp_f / max_exact) / np.log(MAX_DISTANCE / max_exact) * (nb - max_exact)
    ).astype(jnp.int32)
    large = jnp.minimum(large, jnp.full_like(large, nb - 1))
    return rel_buckets + jnp.where(is_small, ad, large)


def _fused_kernel(table_ref, out_ref, skew_ref, sem):
    bucket = _compute_line()
    line = jnp.zeros((NUM_HEADS, LINE_PAD), jnp.float32)
    for b in range(NUM_BUCKETS):
        val = table_ref[b, :][:, None]
        line = jnp.where(bucket == b, val, line)

    copies = []
    for half in range(2):
        for r in range(half * ROW_BLOCK, (half + 1) * ROW_BLOCK):
            sh = (SKEW - 1) - r
            skew_ref[:, r, :] = line[:, sh : sh + LINE_LEN]
        for gg in range(S // SKEW):
            g = 2 * gg + half
            off = (S - SKEW) - SKEW * gg
            cp = pltpu.make_async_copy(
                skew_ref.at[
                    :,
                    pl.ds(half * ROW_BLOCK, ROW_BLOCK),
                    pl.ds(off, S),
                ],
                out_ref.at[:, pl.ds(g * ROW_BLOCK, ROW_BLOCK), :],
                sem,
            )
            cp.start()
            copies.append(cp)
    for cp in copies:
        cp.wait()


def kernel(input_ids, attention_mask, bias_table):
    del input_ids, attention_mask
    out = pl.pallas_call(
        _fused_kernel,
        in_specs=[pl.BlockSpec((NUM_BUCKETS, NUM_HEADS), lambda: (0, 0))],
        out_specs=pl.BlockSpec(memory_space=pl.ANY),
        out_shape=jax.ShapeDtypeStruct((NUM_HEADS, S, S), jnp.float32),
        scratch_shapes=[
            pltpu.VMEM((NUM_HEADS, SKEW, LINE_LEN), jnp.float32),
            pltpu.SemaphoreType.DMA,
        ],
    )(bias_table)
    return out[None]


# speedup vs baseline: 4.9280x; 1.0030x over previous
import functools

import jax
import jax.numpy as jnp
import numpy as np
from jax import lax
from jax.experimental import pallas as pl
from jax.experimental.pallas import tpu as pltpu

NUM_BUCKETS = 32
MAX_DISTANCE = 128
NUM_HEADS = 16
S = 2048
SKEW = 128
ROW_BLOCK = 64
LINE_LEN = 3968
LINE_PAD = LINE_LEN + SKEW


def _compute_line():
    k = lax.broadcasted_iota(jnp.int32, (NUM_HEADS, LINE_PAD), 1)
    d = k - (S - 1)
    nb = NUM_BUCKETS // 2
    rel_buckets = (d > 0).astype(jnp.int32) * nb
    ad = jnp.abs(d)
    max_exact = nb // 2
    is_small = ad < max_exact
    rp_f = jnp.maximum(ad, 1).astype(jnp.float32)
    large = max_exact + (
        jnp.log(rp_f / max_exact) / np.log(MAX_DISTANCE / max_exact) * (nb - max_exact)
    ).astype(jnp.int32)
    large = jnp.minimum(large, jnp.full_like(large, nb - 1))
    return rel_buckets + jnp.where(is_small, ad, large)


def _fused_kernel(table_ref, out_ref, skew_ref, sem):
    bucket = _compute_line()
    half_w = LINE_PAD // 2
    left = jnp.zeros((NUM_HEADS, half_w), jnp.float32)
    right = jnp.zeros((NUM_HEADS, half_w), jnp.float32)
    b_left = bucket[:, :half_w]
    b_right = bucket[:, half_w:]
    for b in range(NUM_BUCKETS // 2):
        val = table_ref[b, :][:, None]
        left = jnp.where(b_left == b, val, left)
    for b in range(NUM_BUCKETS // 2, NUM_BUCKETS):
        val = table_ref[b, :][:, None]
        right = jnp.where(b_right == b, val, right)
    line = jnp.concatenate([left, right], axis=1)

    copies = []
    for half in range(2):
        for r in range(half * ROW_BLOCK, (half + 1) * ROW_BLOCK):
            sh = (SKEW - 1) - r
            skew_ref[:, r, :] = line[:, sh : sh + LINE_LEN]
        for gg in range(S // SKEW):
            g = 2 * gg + half
            off = (S - SKEW) - SKEW * gg
            cp = pltpu.make_async_copy(
                skew_ref.at[
                    :,
                    pl.ds(half * ROW_BLOCK, ROW_BLOCK),
                    pl.ds(off, S),
                ],
                out_ref.at[:, pl.ds(g * ROW_BLOCK, ROW_BLOCK), :],
                sem,
            )
            cp.start()
            copies.append(cp)
    for cp in copies:
        cp.wait()


def kernel(input_ids, attention_mask, bias_table):
    del input_ids, attention_mask
    out = pl.pallas_call(
        _fused_kernel,
        in_specs=[pl.BlockSpec((NUM_BUCKETS, NUM_HEADS), lambda: (0, 0))],
        out_specs=pl.BlockSpec(memory_space=pl.ANY),
        out_shape=jax.ShapeDtypeStruct((NUM_HEADS, S, S), jnp.float32),
        scratch_shapes=[
            pltpu.VMEM((NUM_HEADS, SKEW, LINE_LEN), jnp.float32),
            pltpu.SemaphoreType.DMA,
        ],
    )(bias_table)
    return out[None]


# speedup vs baseline: 5.0305x; 1.0208x over previous
import functools

import jax
import jax.numpy as jnp
import numpy as np
from jax import lax
from jax.experimental import pallas as pl
from jax.experimental.pallas import tpu as pltpu

NUM_BUCKETS = 32
MAX_DISTANCE = 128
NUM_HEADS = 16
S = 2048
SKEW = 128
ROW_BLOCK = 64
LINE_LEN = 3968
LINE_PAD = LINE_LEN + SKEW


def _compute_line():
    k = lax.broadcasted_iota(jnp.int32, (NUM_HEADS, LINE_PAD), 1)
    d = k - (S - 1)
    nb = NUM_BUCKETS // 2
    rel_buckets = (d > 0).astype(jnp.int32) * nb
    ad = jnp.abs(d)
    max_exact = nb // 2
    is_small = ad < max_exact
    rp_f = jnp.maximum(ad, 1).astype(jnp.float32)
    large = max_exact + (
        jnp.log(rp_f / max_exact) / np.log(MAX_DISTANCE / max_exact) * (nb - max_exact)
    ).astype(jnp.int32)
    large = jnp.minimum(large, jnp.full_like(large, nb - 1))
    return rel_buckets + jnp.where(is_small, ad, large)


def _fused_kernel(table_ref, out_ref, skew_ref, sem):
    bucket = _compute_line()
    half_w = LINE_PAD // 2
    left = jnp.zeros((NUM_HEADS, half_w), jnp.float32)
    right = jnp.zeros((NUM_HEADS, half_w), jnp.float32)
    b_left = bucket[:, :half_w]
    b_right = bucket[:, half_w:]
    for b in range(NUM_BUCKETS // 2):
        val = table_ref[b, :][:, None]
        left = jnp.where(b_left == b, val, left)
    for b in range(NUM_BUCKETS // 2, NUM_BUCKETS):
        val = table_ref[b, :][:, None]
        right = jnp.where(b_right == b, val, right)
    line = jnp.concatenate([left, right], axis=1)

    copies = []

    def fire(rstart, nrows, off, out_row0):
        cp = pltpu.make_async_copy(
            skew_ref.at[:, pl.ds(rstart, nrows), pl.ds(off, S)],
            out_ref.at[:, pl.ds(out_row0, nrows), :],
            sem,
        )
        cp.start()
        copies.append(cp)

    for half in range(2):
        for q in range(ROW_BLOCK // 8):
            for r in range(half * ROW_BLOCK + q * 8, half * ROW_BLOCK + q * 8 + 8):
                sh = (SKEW - 1) - r
                skew_ref[:, r, :] = line[:, sh : sh + LINE_LEN]
            if half == 0:
                fire(q * 8, 8, S - SKEW, q * 8)
        for gg in range(1 - half, S // SKEW):
            g = 2 * gg + half
            off = (S - SKEW) - SKEW * gg
            fire(half * ROW_BLOCK, ROW_BLOCK, off, g * ROW_BLOCK)
    for cp in copies:
        cp.wait()


def kernel(input_ids, attention_mask, bias_table):
    del input_ids, attention_mask
    out = pl.pallas_call(
        _fused_kernel,
        in_specs=[pl.BlockSpec((NUM_BUCKETS, NUM_HEADS), lambda: (0, 0))],
        out_specs=pl.BlockSpec(memory_space=pl.ANY),
        out_shape=jax.ShapeDtypeStruct((NUM_HEADS, S, S), jnp.float32),
        scratch_shapes=[
            pltpu.VMEM((NUM_HEADS, SKEW, LINE_LEN), jnp.float32),
            pltpu.SemaphoreType.DMA,
        ],
    )(bias_table)
    return out[None]
